# baseline (speedup 1.0000x reference)
import functools

import jax
import jax.numpy as jnp
from jax import lax
from jax.experimental import pallas as pl
from jax.experimental.pallas import tpu as pltpu
from jax.experimental.pallas import tpu_sc as plsc

_INFO = plsc.get_sparse_core_info()
_NC, _NS, _L = _INFO.num_cores, _INFO.num_subcores, _INFO.num_lanes
_NW = _NC * _NS

_CR = 8


@functools.lru_cache(maxsize=None)
def _build(b, s, d):
    rows_per_w = b // _NW
    chunks = rows_per_w // _CR
    assert chunks % 2 == 0

    mesh = plsc.VectorSubcoreMesh(core_axis_name="c", subcore_axis_name="s")

    @functools.partial(
        pl.kernel,
        out_type=jax.ShapeDtypeStruct((b, s, d), jnp.float32),
        mesh=mesh,
        scratch_types=[
            pltpu.VMEM((rows_per_w, s), jnp.int32),
            pltpu.VMEM((2, _CR, s, d), jnp.float32),
            pltpu.SemaphoreType.DMA,
            pltpu.SemaphoreType.DMA,
            pltpu.SemaphoreType.DMA,
            pltpu.SemaphoreType.DMA,
        ],
        compiler_params=pltpu.CompilerParams(use_tc_tiling_on_sc=False),
    )
    def k(tid_hbm, table_hbm, out_hbm, idx_v, rows_v, g0, g1, w0, w1):
        gsem = (g0, g1)
        wsem = (w0, w1)
        wid = lax.axis_index("c") * _NS + lax.axis_index("s")
        base_row = wid * rows_per_w

        pltpu.sync_copy(tid_hbm.at[pl.ds(base_row, rows_per_w)], idx_v)

        def fire_gather(g, buf):
            for j in range(_CR):
                pltpu.async_copy(
                    table_hbm.at[idx_v.at[g * _CR + j]],
                    rows_v.at[buf].at[j],
                    gsem[buf],
                )

        def wait_gather(buf):
            pltpu.make_async_copy(
                out_hbm.at[pl.ds(0, _CR)], rows_v.at[buf], gsem[buf]
            ).wait()

        def fire_writeback(g, buf):
            pltpu.async_copy(
                rows_v.at[buf],
                out_hbm.at[pl.ds(base_row + g * _CR, _CR)],
                wsem[buf],
            )

        def wait_writeback(buf):
            pltpu.make_async_copy(
                rows_v.at[buf], out_hbm.at[pl.ds(0, _CR)], wsem[buf]
            ).wait()

        fire_gather(0, 0)

        def body(i, carry):
            for buf in range(2):
                g = i * 2 + buf
                nb = 1 - buf

                def _wait_prev_wb():
                    wait_writeback(nb)

                if buf == 1:
                    _wait_prev_wb()
                else:
                    pl.when(g >= 1)(_wait_prev_wb)

                def _fire_next():
                    fire_gather(g + 1, nb)

                pl.when(g + 1 < chunks)(_fire_next)
                wait_gather(buf)
                fire_writeback(g, buf)
            return carry

        lax.fori_loop(0, chunks // 2, body, 0)
        wait_writeback((chunks - 1) % 2)

    return k


def kernel(token_ids, weight):
    b, s = token_ids.shape
    d = weight.shape[1]
    return _build(b, s, d)(token_ids.astype(jnp.int32), weight)
